# Initial kernel scaffold; baseline (speedup 1.0000x reference)
import jax
import jax.numpy as jnp
from jax.experimental import pallas as pl


def kernel(c1_w, c1_b, c2_w, c2_b, fc1_w, fc1_b, fc2_w, fc2_b, out_w, out_b, x):
    raise NotImplementedError("write your pallas kernel here")


# speedup vs baseline: 175.0081x; 175.0081x over previous
import numpy as np
import jax
import jax.numpy as jnp
from jax.experimental import pallas as pl
from jax.experimental.pallas import tpu as pltpu

LANES = 128
TB = 512
OP_DT = jnp.bfloat16


def _conv1_maps():
    idx = np.zeros((140, 256), np.int32)
    msk = np.zeros((140, 256), np.float32)
    for p in range(2):
        for o in range(6):
            for jp in range(12):
                col = p * 128 + o * 12 + jp
                j = 2 * jp + p
                for di in range(5):
                    for dj in range(5):
                        idx[di * 28 + j + dj, col] = (di * 5 + dj) * LANES + o
                        msk[di * 28 + j + dj, col] = 1.0
    return idx, msk


def _conv2_maps():
    idx = np.zeros((640, 256), np.int32)
    msk = np.zeros((640, 256), np.float32)
    for p in range(2):
        for o2 in range(12):
            for j2p in range(4):
                col = p * 128 + o2 * 4 + j2p
                j = 2 * j2p + p
                for di in range(5):
                    for c in range(6):
                        for dj in range(5):
                            row = di * 128 + c * 12 + j + dj
                            idx[row, col] = ((di * 5 + dj) * 6 + c) * LANES + o2
                            msk[row, col] = 1.0
    return idx, msk


def _fc1_maps():
    rows = np.zeros((512,), np.int32)
    msk = np.zeros((512, 1), np.float32)
    for h in range(4):
        for c in range(12):
            for w in range(4):
                rows[h * 128 + c * 4 + w] = h * 48 + w * 12 + c
                msk[h * 128 + c * 4 + w, 0] = 1.0
    return rows, msk


def _bias_maps(n_ch, n_sp):
    idx = np.zeros((LANES,), np.int32)
    msk = np.zeros((LANES,), np.float32)
    for o in range(n_ch):
        for jp in range(n_sp):
            idx[o * n_sp + jp] = o
            msk[o * n_sp + jp] = 1.0
    return idx, msk


_IDX1, _MSK1 = _conv1_maps()
_IDX2, _MSK2 = _conv2_maps()
_ROWF, _MSKF = _fc1_maps()
_B1I, _B1M = _bias_maps(6, 12)
_B2I, _B2M = _bias_maps(12, 4)


def _lenet_kernel(x_ref, w1_ref, b1_ref, w2_ref, b2_ref, wf_ref, bf_ref,
                  wg_ref, bg_ref, wo_ref, bo_ref, o_ref, s1, s2):
    xb = x_ref[...].astype(OP_DT)
    w1 = w1_ref[...]
    for r in range(12):
        m = None
        for i in (2 * r, 2 * r + 1):
            d = jnp.dot(xb[:, i * 28:i * 28 + 140], w1,
                        preferred_element_type=jnp.float32)
            mm = jnp.maximum(d[:, :LANES], d[:, LANES:])
            m = mm if m is None else jnp.maximum(m, mm)
        s1[:, r * LANES:(r + 1) * LANES] = (
            jnp.maximum(m + b1_ref[...], 0.0).astype(OP_DT))

    w2 = w2_ref[...]
    for r in range(4):
        m = None
        for i in (2 * r, 2 * r + 1):
            d = jnp.dot(s1[:, i * LANES:i * LANES + 640], w2,
                        preferred_element_type=jnp.float32)
            mm = jnp.maximum(d[:, :LANES], d[:, LANES:])
            m = mm if m is None else jnp.maximum(m, mm)
        s2[:, r * LANES:(r + 1) * LANES] = (
            jnp.maximum(m + b2_ref[...], 0.0).astype(OP_DT))

    h = jnp.dot(s2[...], wf_ref[...], preferred_element_type=jnp.float32)
    h = jnp.maximum(h + bf_ref[...], 0.0).astype(OP_DT)
    h = jnp.dot(h, wg_ref[...], preferred_element_type=jnp.float32)
    h = jnp.maximum(h + bg_ref[...], 0.0).astype(OP_DT)
    o = jnp.dot(h, wo_ref[...], preferred_element_type=jnp.float32)
    o_ref[...] = o + bo_ref[...]


def _full(shape):
    return pl.BlockSpec(shape, lambda i: (0,) * len(shape))


def kernel(c1_w, c1_b, c2_w, c2_b, fc1_w, fc1_b, fc2_w, fc2_b, out_w, out_b, x):
    B = x.shape[0]
    tb = TB if B % TB == 0 else B
    x2 = x.reshape(B, 28 * 28)

    w1 = (c1_w.reshape(-1)[_IDX1] * _MSK1).astype(OP_DT)
    w2 = (c2_w.reshape(-1)[_IDX2] * _MSK2).astype(OP_DT)
    wf = (fc1_w[_ROWF] * _MSKF).astype(OP_DT)
    b1 = (c1_b.reshape(-1)[_B1I] * _B1M).reshape(1, LANES)
    b2 = (c2_b.reshape(-1)[_B2I] * _B2M).reshape(1, LANES)
    wg = fc2_w.astype(OP_DT)
    wo = out_w.astype(OP_DT)

    out = pl.pallas_call(
        _lenet_kernel,
        out_shape=jax.ShapeDtypeStruct((B, LANES), jnp.float32),
        grid=(B // tb,),
        in_specs=[pl.BlockSpec((tb, 28 * 28), lambda i: (i, 0)),
                  _full(w1.shape), _full(b1.shape),
                  _full(w2.shape), _full(b2.shape),
                  _full(wf.shape), _full(fc1_b.shape),
                  _full(wg.shape), _full(fc2_b.shape),
                  _full(wo.shape), _full(out_b.shape)],
        out_specs=pl.BlockSpec((tb, LANES), lambda i: (i, 0)),
        scratch_shapes=[pltpu.VMEM((tb, 12 * LANES), OP_DT),
                        pltpu.VMEM((tb, 4 * LANES), OP_DT)],
        compiler_params=pltpu.CompilerParams(
            dimension_semantics=("parallel",)),
    )(x2, w1, b1, w2, b2, wf, fc1_b, wg, fc2_b, wo, out_b)
    return out[:, :10]


# speedup vs baseline: 2223.3366x; 12.7042x over previous
import numpy as np
import jax
import jax.numpy as jnp
from jax.experimental import pallas as pl
from jax.experimental.pallas import tpu as pltpu

LANES = 128
TB = 512
OP_DT = jnp.bfloat16


def _conv1_tabs():
    e1 = np.zeros((128, 256), np.float32)
    for p in range(2):
        for o in range(6):
            for jp in range(12):
                e1[o, p * 128 + o * 12 + jp] = 1.0
    c1 = np.zeros((5, 140, 32), np.float32)
    m1 = np.zeros((5, 140, 256), np.float32)
    for dj in range(5):
        for di in range(5):
            for jj in range(28):
                c1[dj, di * 28 + jj, di * 5 + dj] = 1.0
        for p in range(2):
            for o in range(6):
                for jp in range(12):
                    col = p * 128 + o * 12 + jp
                    jj = 2 * jp + p + dj
                    for di in range(5):
                        m1[dj, di * 28 + jj, col] = 1.0
    return e1, c1, m1


def _conv2_tabs():
    e2 = np.zeros((128, 256), np.float32)
    for p in range(2):
        for o2 in range(12):
            for j2p in range(4):
                e2[o2, p * 128 + o2 * 4 + j2p] = 1.0
    c2 = np.zeros((5, 640, 152), np.float32)
    m2 = np.zeros((5, 640, 256), np.float32)
    for dj in range(5):
        for di in range(5):
            for c in range(6):
                for ww in range(12):
                    c2[dj, di * 128 + c * 12 + ww, (di * 5 + dj) * 6 + c] = 1.0
        for p in range(2):
            for o2 in range(12):
                for j2p in range(4):
                    col = p * 128 + o2 * 4 + j2p
                    ww = 2 * j2p + p + dj
                    for di in range(5):
                        for c in range(6):
                            m2[dj, di * 128 + c * 12 + ww, col] = 1.0
    return e2, c2, m2


def _fc1_tab():
    pf = np.zeros((512, 192), np.float32)
    for h in range(4):
        for c in range(12):
            for w in range(4):
                pf[h * 128 + c * 4 + w, h * 48 + w * 12 + c] = 1.0
    return pf


_E1, _C1, _M1 = _conv1_tabs()
_E2, _C2, _M2 = _conv2_tabs()
_PF = _fc1_tab()


def _prep_kernel(c1w, c1b, c2w, c2b, fc1w, e1, c1t, m1, e2, c2t, m2, pf,
                 w1_o, b1_o, w2_o, b2_o, wf_o):
    f32 = jnp.float32
    v1 = jnp.dot(c1w[...], e1[...], preferred_element_type=f32)
    w1 = jnp.zeros((140, 256), f32)
    for dj in range(5):
        w1 = w1 + jnp.dot(c1t[dj], v1, preferred_element_type=f32) * m1[dj]
    w1_o[...] = w1.astype(w1_o.dtype)
    b1_o[...] = jnp.dot(c1b[...], e1[...], preferred_element_type=f32)

    v2 = jnp.dot(c2w[...], e2[...], preferred_element_type=f32)
    w2 = jnp.zeros((640, 256), f32)
    for dj in range(5):
        w2 = w2 + jnp.dot(c2t[dj], v2, preferred_element_type=f32) * m2[dj]
    w2_o[...] = w2.astype(w2_o.dtype)
    b2_o[...] = jnp.dot(c2b[...], e2[...], preferred_element_type=f32)

    wf_o[...] = jnp.dot(pf[...], fc1w[...],
                        preferred_element_type=f32).astype(wf_o.dtype)


def _full(shape):
    return pl.BlockSpec(shape, lambda: (0,) * len(shape))


def _prep(c1_w, c1_b, c2_w, c2_b, fc1_w):
    c1p = jnp.pad(c1_w, ((0, 7), (0, 0)))
    c2p = jnp.pad(c2_w, ((0, 2), (0, 0)))
    outs = pl.pallas_call(
        _prep_kernel,
        out_shape=[jax.ShapeDtypeStruct((140, 256), OP_DT),
                   jax.ShapeDtypeStruct((1, 256), jnp.float32),
                   jax.ShapeDtypeStruct((640, 256), OP_DT),
                   jax.ShapeDtypeStruct((1, 256), jnp.float32),
                   jax.ShapeDtypeStruct((512, 128), OP_DT)],
        in_specs=[_full((32, 128)), _full((1, 128)),
                  _full((152, 128)), _full((1, 128)),
                  _full((192, 128)),
                  _full((128, 256)), _full((5, 140, 32)), _full((5, 140, 256)),
                  _full((128, 256)), _full((5, 640, 152)), _full((5, 640, 256)),
                  _full((512, 192))],
        out_specs=[_full((140, 256)), _full((1, 256)),
                   _full((640, 256)), _full((1, 256)),
                   _full((512, 128))],
    )(c1p, c1_b, c2p, c2_b, fc1_w,
      _E1, _C1, _M1, _E2, _C2, _M2, _PF)
    w1, b1, w2, b2, wf = outs
    return w1, b1[:, :LANES], w2, b2[:, :LANES], wf


def _lenet_kernel(x_ref, w1_ref, b1_ref, w2_ref, b2_ref, wf_ref, bf_ref,
                  wg_ref, bg_ref, wo_ref, bo_ref, o_ref, s1, s2):
    tb = x_ref.shape[0]
    xb = x_ref[...].astype(OP_DT).reshape(tb, 784)
    w1 = w1_ref[...]
    for r in range(12):
        m = None
        for i in (2 * r, 2 * r + 1):
            d = jnp.dot(xb[:, i * 28:i * 28 + 140], w1,
                        preferred_element_type=jnp.float32)
            mm = jnp.maximum(d[:, :LANES], d[:, LANES:])
            m = mm if m is None else jnp.maximum(m, mm)
        s1[:, r * LANES:(r + 1) * LANES] = (
            jnp.maximum(m + b1_ref[...], 0.0).astype(OP_DT))

    w2 = w2_ref[...]
    for r in range(4):
        m = None
        for i in (2 * r, 2 * r + 1):
            d = jnp.dot(s1[:, i * LANES:i * LANES + 640], w2,
                        preferred_element_type=jnp.float32)
            mm = jnp.maximum(d[:, :LANES], d[:, LANES:])
            m = mm if m is None else jnp.maximum(m, mm)
        s2[:, r * LANES:(r + 1) * LANES] = (
            jnp.maximum(m + b2_ref[...], 0.0).astype(OP_DT))

    h = jnp.dot(s2[...], wf_ref[...], preferred_element_type=jnp.float32)
    h = jnp.maximum(h + bf_ref[...], 0.0).astype(OP_DT)
    h = jnp.dot(h, wg_ref[...], preferred_element_type=jnp.float32)
    h = jnp.maximum(h + bg_ref[...], 0.0).astype(OP_DT)
    o = jnp.dot(h, wo_ref[...], preferred_element_type=jnp.float32)
    o_ref[...] = o + bo_ref[...]


def kernel(c1_w, c1_b, c2_w, c2_b, fc1_w, fc1_b, fc2_w, fc2_b, out_w, out_b, x):
    B = x.shape[0]
    tb = TB if B % TB == 0 else B
    x3 = x.reshape(B, 28, 28)

    w1, b1, w2, b2, wf = _prep(c1_w, c1_b, c2_w, c2_b, fc1_w)
    wg = fc2_w.astype(OP_DT)
    wo = out_w.astype(OP_DT)

    out = pl.pallas_call(
        _lenet_kernel,
        out_shape=jax.ShapeDtypeStruct((B, LANES), jnp.float32),
        grid=(B // tb,),
        in_specs=[pl.BlockSpec((tb, 28, 28), lambda i: (i, 0, 0)),
                  pl.BlockSpec(w1.shape, lambda i: (0, 0)),
                  pl.BlockSpec(b1.shape, lambda i: (0, 0)),
                  pl.BlockSpec(w2.shape, lambda i: (0, 0)),
                  pl.BlockSpec(b2.shape, lambda i: (0, 0)),
                  pl.BlockSpec(wf.shape, lambda i: (0, 0)),
                  pl.BlockSpec(fc1_b.shape, lambda i: (0, 0)),
                  pl.BlockSpec(wg.shape, lambda i: (0, 0)),
                  pl.BlockSpec(fc2_b.shape, lambda i: (0, 0)),
                  pl.BlockSpec(wo.shape, lambda i: (0, 0)),
                  pl.BlockSpec(out_b.shape, lambda i: (0, 0))],
        out_specs=pl.BlockSpec((tb, LANES), lambda i: (i, 0)),
        scratch_shapes=[pltpu.VMEM((tb, 12 * LANES), OP_DT),
                        pltpu.VMEM((tb, 4 * LANES), OP_DT)],
        compiler_params=pltpu.CompilerParams(
            dimension_semantics=("parallel",)),
    )(x3, w1, b1, w2, b2, wf, fc1_b, wg, fc2_b, wo, out_b)
    return out[:, :10]
